# Initial kernel scaffold; baseline (speedup 1.0000x reference)
import jax
import jax.numpy as jnp
from jax.experimental import pallas as pl


def kernel(x, edge_index, edge_attr, batch, params):
    raise NotImplementedError("write your pallas kernel here")


# speedup vs baseline: 5.6860x; 5.6860x over previous
import functools

import jax
import jax.numpy as jnp
from jax import lax
from jax.experimental import pallas as pl
from jax.experimental.pallas import tpu as pltpu
from jax.experimental.pallas import tpu_sc as plsc

N = 10000
E = 160000
IN_DIM = 128
EMB = 256
HEADS = 4
LAYERS = 3
EDGE_DIM = 16
B = 64

NTILE = 32
NPT = 320
NPAD = NTILE * NPT
EP = 170240
EP_A = EP + 128
EPL = EP + 4096
RB = 80
NRB = N // RB
CE_A = 128
CE_B = 64
DEN = NPT * HEADS


def _mesh():
    return plsc.VectorSubcoreMesh(
        core_axis_name="c", subcore_axis_name="s", num_cores=2, num_subcores=16)


def _h0_body(x_ref, w_ref, b_ref, o_ref):
    v = jnp.dot(x_ref[...], w_ref[...], preferred_element_type=jnp.float32)
    o_ref[...] = jnp.maximum(v + b_ref[...], 0.0)


def _tc_h0(x, w0, b0):
    return pl.pallas_call(
        _h0_body,
        grid=(NRB,),
        in_specs=[
            pl.BlockSpec((RB, IN_DIM), lambda i: (i, 0)),
            pl.BlockSpec((IN_DIM, EMB), lambda i: (0, 0)),
            pl.BlockSpec((1, EMB), lambda i: (0, 0)),
        ],
        out_specs=pl.BlockSpec((RB, EMB), lambda i: (i, 0)),
        out_shape=jax.ShapeDtypeStruct((N, EMB), jnp.float32),
    )(x, w0, b0.reshape(1, EMB))


def _xh_body(h_ref, w_ref, o_ref):
    o_ref[0] = jnp.dot(h_ref[...], w_ref[...], preferred_element_type=jnp.float32)


def _tc_xh(h, w):
    return pl.pallas_call(
        _xh_body,
        grid=(HEADS, NRB),
        in_specs=[
            pl.BlockSpec((RB, EMB), lambda hd, rb: (rb, 0)),
            pl.BlockSpec((EMB, EMB), lambda hd, rb: (0, hd)),
        ],
        out_specs=pl.BlockSpec((1, RB, EMB), lambda hd, rb: (hd, rb, 0)),
        out_shape=jax.ShapeDtypeStruct((HEADS, N, EMB), jnp.float32),
    )(h, w)


def _s_body(h_ref, ws_ref, o_ref):
    o_ref[...] = jnp.dot(h_ref[...], ws_ref[...], preferred_element_type=jnp.float32)


def _tc_scores(h, ws):
    return pl.pallas_call(
        _s_body,
        grid=(NRB,),
        in_specs=[
            pl.BlockSpec((RB, EMB), lambda i: (i, 0)),
            pl.BlockSpec((EMB, 2 * HEADS), lambda i: (0, 0)),
        ],
        out_specs=pl.BlockSpec((RB, 2 * HEADS), lambda i: (i, 0)),
        out_shape=jax.ShapeDtypeStruct((N, 2 * HEADS), jnp.float32),
    )(h, ws)


def _prep_w_body(w_ref, asrc_ref, adst_ref, o_ref):
    w3 = w_ref[...].reshape(EMB, HEADS, EMB)
    ws = jnp.sum(w3 * asrc_ref[...][None, :, :], axis=2)
    wd = jnp.sum(w3 * adst_ref[...][None, :, :], axis=2)
    o_ref[...] = jnp.concatenate([ws, wd], axis=1)


def _tc_prep_w(w, att_src, att_dst):
    return pl.pallas_call(
        _prep_w_body,
        out_shape=jax.ShapeDtypeStruct((EMB, 2 * HEADS), jnp.float32),
    )(w, att_src, att_dst)


def _prep_we_body(we_ref, ae_ref, o_ref):
    w3 = we_ref[...].reshape(EDGE_DIM, HEADS, EMB)
    o_ref[...] = jnp.sum(w3 * ae_ref[...][None, :, :], axis=2)


def _tc_prep_we(we, att_e):
    return pl.pallas_call(
        _prep_we_body,
        out_shape=jax.ShapeDtypeStruct((EDGE_DIM, HEADS), jnp.float32),
    )(we, att_e)


def _se_body(ea_ref, w1_ref, w2_ref, w3_ref, o1_ref, o2_ref, o3_ref,
             c1_ref, c2_ref, c3_ref):
    step = pl.program_id(0)
    ea = ea_ref[...]
    for w_ref, o_ref, c_ref in ((w1_ref, o1_ref, c1_ref),
                                (w2_ref, o2_ref, c2_ref),
                                (w3_ref, o3_ref, c3_ref)):
        v = jnp.dot(ea, w_ref[...], preferred_element_type=jnp.float32)
        o_ref[...] = v.T
        s = jnp.broadcast_to(jnp.sum(v, axis=0, keepdims=True), (8, HEADS))

        @pl.when(step == 0)
        def _():
            c_ref[...] = s

        @pl.when(step != 0)
        def _():
            c_ref[...] = c_ref[...] + s


def _tc_se(edge_attr, we1, we2, we3):
    eb = 128
    neb = E // eb
    return pl.pallas_call(
        _se_body,
        grid=(neb,),
        in_specs=[
            pl.BlockSpec((eb, EDGE_DIM), lambda i: (i, 0)),
            pl.BlockSpec((EDGE_DIM, HEADS), lambda i: (0, 0)),
            pl.BlockSpec((EDGE_DIM, HEADS), lambda i: (0, 0)),
            pl.BlockSpec((EDGE_DIM, HEADS), lambda i: (0, 0)),
        ],
        out_specs=[
            pl.BlockSpec((HEADS, eb), lambda i: (0, i)),
            pl.BlockSpec((HEADS, eb), lambda i: (0, i)),
            pl.BlockSpec((HEADS, eb), lambda i: (0, i)),
            pl.BlockSpec((8, HEADS), lambda i: (0, 0)),
            pl.BlockSpec((8, HEADS), lambda i: (0, 0)),
            pl.BlockSpec((8, HEADS), lambda i: (0, 0)),
        ],
        out_shape=[
            jax.ShapeDtypeStruct((HEADS, E), jnp.float32),
            jax.ShapeDtypeStruct((HEADS, E), jnp.float32),
            jax.ShapeDtypeStruct((HEADS, E), jnp.float32),
            jax.ShapeDtypeStruct((8, HEADS), jnp.float32),
            jax.ShapeDtypeStruct((8, HEADS), jnp.float32),
            jax.ShapeDtypeStruct((8, HEADS), jnp.float32),
        ],
    )(edge_attr, we1, we2, we3)


def _se_fill_body(c1_ref, c2_ref, c3_ref, o1_ref, o2_ref, o3_ref):
    col = lax.broadcasted_iota(jnp.int32, (HEADS, EP_A - E), 1)
    for c_ref, o_ref in ((c1_ref, o1_ref), (c2_ref, o2_ref), (c3_ref, o3_ref)):
        loopval = c_ref[...][0:1, 0:HEADS].T * (1.0 / E)
        o_ref[...] = jnp.where(col < (EP - E),
                               jnp.broadcast_to(loopval, (HEADS, EP_A - E)),
                               -1e30)


def _tc_se_fill(c1, c2, c3):
    sh = jax.ShapeDtypeStruct((HEADS, EP_A - E), jnp.float32)
    return pl.pallas_call(
        _se_fill_body,
        out_shape=[sh, sh, sh],
    )(c1, c2, c3)


def _bn_stats_body(m_ref, o_ref):
    step = pl.program_id(0)
    m = m_ref[...]
    s = jnp.sum(m, axis=0, keepdims=True)
    q = jnp.sum(m * m, axis=0, keepdims=True)
    blk = jnp.concatenate([s, q, jnp.zeros((6, EMB), jnp.float32)], axis=0)

    @pl.when(step == 0)
    def _():
        o_ref[...] = blk

    @pl.when(step != 0)
    def _():
        o_ref[...] = o_ref[...] + blk


def _tc_bn_stats(msg):
    return pl.pallas_call(
        _bn_stats_body,
        grid=(NRB,),
        in_specs=[pl.BlockSpec((RB, EMB), lambda i: (i, 0))],
        out_specs=pl.BlockSpec((8, EMB), lambda i: (0, 0)),
        out_shape=jax.ShapeDtypeStruct((8, EMB), jnp.float32),
    )(msg)


def _bn_apply_body(m_ref, hp_ref, st_ref, g_ref, bt_ref, o_ref):
    st = st_ref[...]
    mu = st[0:1, :] * (1.0 / N)
    var = st[1:2, :] * (1.0 / N) - mu * mu
    inv = lax.rsqrt(var + 1e-5)
    hn = (m_ref[...] - mu) * (inv * g_ref[...]) + bt_ref[...]
    o_ref[...] = hp_ref[...] + jnp.maximum(hn, 0.0)


def _tc_bn_apply(msg, h_prev, stats, gamma, beta):
    return pl.pallas_call(
        _bn_apply_body,
        grid=(NRB,),
        in_specs=[
            pl.BlockSpec((RB, EMB), lambda i: (i, 0)),
            pl.BlockSpec((RB, EMB), lambda i: (i, 0)),
            pl.BlockSpec((8, EMB), lambda i: (0, 0)),
            pl.BlockSpec((1, EMB), lambda i: (0, 0)),
            pl.BlockSpec((1, EMB), lambda i: (0, 0)),
        ],
        out_specs=pl.BlockSpec((RB, EMB), lambda i: (i, 0)),
        out_shape=jax.ShapeDtypeStruct((N, EMB), jnp.float32),
    )(msg, h_prev, stats, gamma.reshape(1, EMB), beta.reshape(1, EMB))


def _pool_body(h1_ref, h2_ref, h3_ref, b_ref, o1_ref, o2_ref, o3_ref):
    step = pl.program_id(0)
    bvec = b_ref[...][:, 0]
    oh = (lax.broadcasted_iota(jnp.int32, (B, RB), 0) == bvec[None, :])
    oh = oh.astype(jnp.float32)
    for h_ref, o_ref in ((h1_ref, o1_ref), (h2_ref, o2_ref), (h3_ref, o3_ref)):
        v = jnp.dot(oh, h_ref[...], preferred_element_type=jnp.float32)

        @pl.when(step == 0)
        def _():
            o_ref[...] = v

        @pl.when(step != 0)
        def _():
            o_ref[...] = o_ref[...] + v


def _tc_pool(h1, h2, h3, batch):
    sh = jax.ShapeDtypeStruct((B, EMB), jnp.float32)
    return pl.pallas_call(
        _pool_body,
        grid=(NRB,),
        in_specs=[
            pl.BlockSpec((RB, EMB), lambda i: (i, 0)),
            pl.BlockSpec((RB, EMB), lambda i: (i, 0)),
            pl.BlockSpec((RB, EMB), lambda i: (i, 0)),
            pl.BlockSpec((RB, 1), lambda i: (i, 0)),
        ],
        out_specs=[
            pl.BlockSpec((B, EMB), lambda i: (0, 0)),
            pl.BlockSpec((B, EMB), lambda i: (0, 0)),
            pl.BlockSpec((B, EMB), lambda i: (0, 0)),
        ],
        out_shape=[sh, sh, sh],
    )(h1, h2, h3, batch.reshape(N, 1))


def _final_body(p1_ref, p2_ref, p3_ref, wg_ref, bg_ref, o_ref):
    p1, p2, p3 = p1_ref[...], p2_ref[...], p3_ref[...]
    zs = jnp.concatenate([p1, p2, p3], axis=1)
    g = jnp.dot(zs, wg_ref[...], preferred_element_type=jnp.float32) + bg_ref[...]
    g = g - jnp.max(g, axis=1, keepdims=True)
    eg = jnp.exp(g)
    g = eg / jnp.sum(eg, axis=1, keepdims=True)
    o_ref[...] = p1 * g[:, 0:1] + p2 * g[:, 1:2] + p3 * g[:, 2:3]


def _tc_final(p1, p2, p3, wg, bg):
    return pl.pallas_call(
        _final_body,
        out_shape=jax.ShapeDtypeStruct((B, EMB), jnp.float32),
    )(p1, p2, p3, wg, bg.reshape(1, LAYERS))


def _sc_alpha_body(s_hbm, se_hbm, perm_hbm, srcs_hbm, dsts_hbm, starts_hbm,
                   alpha_hbm,
                   s_v, den_v, inv_v, perm_v, src_v, dst_v, se_v, al_v,
                   starts_v, sem):
    wid = lax.axis_index("s") * 2 + lax.axis_index("c")
    nb = wid * NPT
    pltpu.sync_copy(starts_hbm, starts_v)
    pltpu.sync_copy(s_hbm, s_v)
    wid16 = jnp.full((16,), wid, jnp.int32)
    estart = plsc.load_gather(starts_v, [wid16])[0]
    eend = plsc.load_gather(starts_v, [wid16 + 1])[0]
    iota = lax.iota(jnp.int32, 16)
    zero16 = jnp.zeros((16,), jnp.float32)

    def zero_den(i, _):
        den_v[pl.ds(i * 16, 16)] = zero16
        return 0

    lax.fori_loop(0, 16 * DEN // 16, zero_den, 0)

    def load_chunk(base):
        c1 = pltpu.async_copy(perm_hbm.at[pl.ds(base, CE_A)], perm_v, sem)
        c2 = pltpu.async_copy(srcs_hbm.at[pl.ds(base, CE_A)], src_v, sem)
        c3 = pltpu.async_copy(dsts_hbm.at[pl.ds(base, CE_A)], dst_v, sem)
        c1.wait()
        c2.wait()
        c3.wait()
        cs = [pltpu.async_copy(se_hbm.at[h].at[perm_v], se_v.at[h], sem)
              for h in range(HEADS)]
        for c in cs:
            c.wait()

    def groups():
        for g in range(CE_A // 16):
            src16 = src_v[pl.ds(g * 16, 16)]
            dst16 = dst_v[pl.ds(g * 16, 16)]
            dl16 = jnp.clip(dst16 - nb, 0, NPT - 1)
            sidx = src16 * 8
            didx = jnp.minimum(dst16, N - 1) * 8 + HEADS
            yield g, sidx, didx, dl16

    def ex_of(g, sidx, didx, h):
        ss = plsc.load_gather(s_v, [sidx + h])
        sd = plsc.load_gather(s_v, [didx + h])
        se16 = se_v[h, pl.ds(g * 16, 16)]
        a = ss + sd + se16
        a = jnp.where(a >= 0, a, a * 0.2)
        return jnp.exp(a)

    nchunk = (eend - estart) // CE_A

    def pass1(i, _):
        base = pl.multiple_of(estart + i * CE_A, CE_A)
        load_chunk(base)
        for g, sidx, didx, dl16 in groups():
            for h in range(HEADS):
                ex = ex_of(g, sidx, didx, h)
                plsc.addupdate_scatter(
                    den_v, [iota * DEN + dl16 * HEADS + h], ex)
        return 0

    lax.fori_loop(0, nchunk, pass1, 0)

    def red(j, _):
        acc = den_v[pl.ds(j * 16, 16)]
        for l in range(1, 16):
            acc = acc + den_v[pl.ds(l * DEN + j * 16, 16)]
        inv_v[pl.ds(j * 16, 16)] = 0.25 / (acc + 1e-16)
        return 0

    lax.fori_loop(0, DEN // 16, red, 0)

    def pass2(i, _):
        base = pl.multiple_of(estart + i * CE_A, CE_A)
        load_chunk(base)
        for g, sidx, didx, dl16 in groups():
            for h in range(HEADS):
                ex = ex_of(g, sidx, didx, h)
                inv16 = plsc.load_gather(inv_v, [dl16 * HEADS + h])
                al_v[h, pl.ds(g * 16, 16)] = ex * inv16
        for h in range(HEADS):
            pltpu.sync_copy(al_v.at[h], alpha_hbm.at[h].at[pl.ds(base, CE_A)])
        return 0

    lax.fori_loop(0, nchunk, pass2, 0)


def _sc_alpha(s_flat, se, perm, srcs, dsts, starts):
    f = functools.partial(
        pl.kernel,
        out_type=jax.ShapeDtypeStruct((HEADS, EPL), jnp.float32),
        mesh=_mesh(),
        compiler_params=pltpu.CompilerParams(
            needs_layout_passes=False, use_tc_tiling_on_sc=False),
        scratch_types=[
            pltpu.VMEM((N * 2 * HEADS,), jnp.float32),
            pltpu.VMEM((16 * DEN,), jnp.float32),
            pltpu.VMEM((DEN,), jnp.float32),
            pltpu.VMEM((CE_A,), jnp.int32),
            pltpu.VMEM((CE_A,), jnp.int32),
            pltpu.VMEM((CE_A,), jnp.int32),
            pltpu.VMEM((HEADS, CE_A), jnp.float32),
            pltpu.VMEM((HEADS, CE_A), jnp.float32),
            pltpu.VMEM((NTILE + 8,), jnp.int32),
            pltpu.SemaphoreType.DMA,
        ],
    )(_sc_alpha_body)
    return f(s_flat, se, perm, srcs, dsts, starts)


def _sc_msg_body(xh0_hbm, xh1_hbm, xh2_hbm, xh3_hbm, alpha_hbm,
                 srcs_hbm, dsts_hbm, starts_hbm,
                 msg_hbm,
                 acc_v, rows_v, src_v, dst_v, al_v,
                 starts_v, sem):
    wid = lax.axis_index("s") * 2 + lax.axis_index("c")
    nb = wid * NPT
    pltpu.sync_copy(starts_hbm, starts_v)
    wid16 = jnp.full((16,), wid, jnp.int32)
    estart = plsc.load_gather(starts_v, [wid16])[0]
    eend = plsc.load_gather(starts_v, [wid16 + 1])[0]
    zero16 = jnp.zeros((16,), jnp.float32)

    def zero_acc(j, _):
        for k in range(EMB // 16):
            acc_v[j, pl.ds(k * 16, 16)] = zero16
        return 0

    lax.fori_loop(0, NPT, zero_acc, 0)

    nchunk = (eend - estart) // CE_B
    xhs = (xh0_hbm, xh1_hbm, xh2_hbm, xh3_hbm)

    def chunk(i, _):
        base = pl.multiple_of(estart + i * CE_B, CE_B)
        c1 = pltpu.async_copy(srcs_hbm.at[pl.ds(base, CE_B)], src_v, sem)
        c2 = pltpu.async_copy(dsts_hbm.at[pl.ds(base, CE_B)], dst_v, sem)
        cs = [pltpu.async_copy(alpha_hbm.at[h].at[pl.ds(base, CE_B)],
                               al_v.at[pl.ds(h * CE_B, CE_B)], sem)
              for h in range(HEADS)]
        c1.wait()
        c2.wait()
        for c in cs:
            c.wait()
        for h in range(HEADS):
            pltpu.async_copy(xhs[h].at[src_v], rows_v, sem).wait()

            def edge(e, _):
                e16 = jnp.full((16,), e, jnp.int32)
                dl = plsc.load_gather(dst_v, [e16])[0] - nb
                a16 = plsc.load_gather(al_v, [e16 + (h * CE_B)])
                for k in range(EMB // 16):
                    sl = pl.ds(k * 16, 16)
                    plsc.addupdate(acc_v.at[dl, sl], a16 * rows_v[e, sl])
                return 0

            lax.fori_loop(0, CE_B, edge, 0)
        return 0

    lax.fori_loop(0, nchunk, chunk, 0)
    pltpu.sync_copy(acc_v, msg_hbm.at[pl.ds(pl.multiple_of(nb, NPT), NPT)])


def _sc_msg(xh0, xh1, xh2, xh3, alpha, srcs, dsts, starts):
    f = functools.partial(
        pl.kernel,
        out_type=jax.ShapeDtypeStruct((NPAD, EMB), jnp.float32),
        mesh=_mesh(),
        compiler_params=pltpu.CompilerParams(
            needs_layout_passes=False, use_tc_tiling_on_sc=False),
        scratch_types=[
            pltpu.VMEM((NPT, EMB), jnp.float32),
            pltpu.VMEM((CE_B, EMB), jnp.float32),
            pltpu.VMEM((CE_B,), jnp.int32),
            pltpu.VMEM((CE_B,), jnp.int32),
            pltpu.VMEM((HEADS * CE_B,), jnp.float32),
            pltpu.VMEM((NTILE + 8,), jnp.int32),
            pltpu.SemaphoreType.DMA,
        ],
    )(_sc_msg_body)
    return f(xh0, xh1, xh2, xh3, alpha, srcs, dsts, starts)


def kernel(x, edge_index, edge_attr, batch, params):
    p = params

    loops = jnp.arange(N, dtype=jnp.int32)
    src_all = jnp.concatenate([edge_index[0], loops])
    dst_all = jnp.concatenate([edge_index[1], loops])
    npad_e = EP - (E + N)
    src_p = jnp.concatenate([src_all, jnp.zeros((npad_e,), jnp.int32)])
    dst_p = jnp.concatenate([dst_all, jnp.full((npad_e,), NPAD - 1, jnp.int32)])
    perm = jnp.argsort(dst_p).astype(jnp.int32)
    src_s = src_p[perm]
    dst_s = dst_p[perm]
    bounds = jnp.searchsorted(
        dst_s, jnp.arange(NTILE + 1, dtype=jnp.int32) * NPT).astype(jnp.int32)
    cnt = bounds[1:] - bounds[:-1]
    cap = ((cnt + CE_A - 1) // CE_A) * CE_A
    astart = jnp.concatenate(
        [jnp.zeros((1,), jnp.int32), jnp.cumsum(cap).astype(jnp.int32)])
    tid = dst_s // NPT
    pos = jnp.arange(EP, dtype=jnp.int32) + (astart[:-1] - bounds[:-1])[tid]
    perm_f = jnp.full((EPL,), EP, jnp.int32).at[pos].set(perm)
    srcs_f = jnp.zeros((EPL,), jnp.int32).at[pos].set(src_s)
    slot_tid = jnp.searchsorted(
        astart[1:], jnp.arange(EPL, dtype=jnp.int32), side='right')
    dinit = jnp.clip((slot_tid + 1) * NPT - 1, 0, NPAD - 1).astype(jnp.int32)
    dsts_f = dinit.at[pos].set(dst_s)
    starts = jnp.concatenate(
        [astart, jnp.full((NTILE + 8 - (NTILE + 1),), astart[-1], jnp.int32)])

    ws_l = [_tc_prep_w(lp['W'], lp['att_src'], lp['att_dst'])
            for lp in p['layers']]
    we_l = [_tc_prep_we(lp['We'], lp['att_e']) for lp in p['layers']]
    se1, se2, se3, c1, c2, c3 = _tc_se(edge_attr, *we_l)
    f1, f2, f3 = _tc_se_fill(c1, c2, c3)
    se_full = [jnp.concatenate([t, f], axis=1)
               for t, f in ((se1, f1), (se2, f2), (se3, f3))]

    h = _tc_h0(x, p['W0'], p['b0'])

    outs = []
    for l, lp in enumerate(p['layers']):
        xh4 = _tc_xh(h, lp['W'])
        s = _tc_scores(h, ws_l[l])
        alpha = _sc_alpha(s.reshape(N * 2 * HEADS), se_full[l],
                          perm_f, srcs_f, dsts_f, starts)
        msg = _sc_msg(xh4[0], xh4[1], xh4[2], xh4[3], alpha,
                      srcs_f, dsts_f, starts)
        stats = _tc_bn_stats(msg)
        h = _tc_bn_apply(msg, h, stats, lp['gamma'], lp['beta'])
        outs.append(h)

    p1, p2, p3 = _tc_pool(outs[0], outs[1], outs[2], batch)
    z = _tc_final(p1, p2, p3, p['Wg'], p['bg'])
    return (z, outs[-1])


# speedup vs baseline: 6.8627x; 1.2069x over previous
import functools

import jax
import jax.numpy as jnp
from jax import lax
from jax.experimental import pallas as pl
from jax.experimental.pallas import tpu as pltpu
from jax.experimental.pallas import tpu_sc as plsc

N = 10000
E = 160000
IN_DIM = 128
EMB = 256
HEADS = 4
LAYERS = 3
EDGE_DIM = 16
B = 64

NTILE = 32
NPT = 320
NPAD = NTILE * NPT
EP = 170240
EP_A = EP + 128
EPL = EP + 8192
RB = 80
NRB = N // RB
CE_A = 128
CE_B = 64
DEN = NPT * HEADS


def _mesh():
    return plsc.VectorSubcoreMesh(
        core_axis_name="c", subcore_axis_name="s", num_cores=2, num_subcores=16)


def _h0_body(x_ref, w_ref, b_ref, o_ref):
    v = jnp.dot(x_ref[...], w_ref[...], preferred_element_type=jnp.float32)
    o_ref[...] = jnp.maximum(v + b_ref[...], 0.0)


def _tc_h0(x, w0, b0):
    return pl.pallas_call(
        _h0_body,
        grid=(NRB,),
        in_specs=[
            pl.BlockSpec((RB, IN_DIM), lambda i: (i, 0)),
            pl.BlockSpec((IN_DIM, EMB), lambda i: (0, 0)),
            pl.BlockSpec((1, EMB), lambda i: (0, 0)),
        ],
        out_specs=pl.BlockSpec((RB, EMB), lambda i: (i, 0)),
        out_shape=jax.ShapeDtypeStruct((N, EMB), jnp.float32),
    )(x, w0, b0.reshape(1, EMB))


def _xh_body(h_ref, w_ref, o_ref):
    o_ref[0] = jnp.dot(h_ref[...], w_ref[...], preferred_element_type=jnp.float32)


def _tc_xh(h, w):
    return pl.pallas_call(
        _xh_body,
        grid=(HEADS, NRB),
        in_specs=[
            pl.BlockSpec((RB, EMB), lambda hd, rb: (rb, 0)),
            pl.BlockSpec((EMB, EMB), lambda hd, rb: (0, hd)),
        ],
        out_specs=pl.BlockSpec((1, RB, EMB), lambda hd, rb: (hd, rb, 0)),
        out_shape=jax.ShapeDtypeStruct((HEADS, N, EMB), jnp.float32),
    )(h, w)


def _s_body(h_ref, ws_ref, o_ref):
    o_ref[...] = jnp.dot(h_ref[...], ws_ref[...], preferred_element_type=jnp.float32)


def _tc_scores(h, ws):
    return pl.pallas_call(
        _s_body,
        grid=(NRB,),
        in_specs=[
            pl.BlockSpec((RB, EMB), lambda i: (i, 0)),
            pl.BlockSpec((EMB, 2 * HEADS), lambda i: (0, 0)),
        ],
        out_specs=pl.BlockSpec((RB, 2 * HEADS), lambda i: (i, 0)),
        out_shape=jax.ShapeDtypeStruct((N, 2 * HEADS), jnp.float32),
    )(h, ws)


def _prep_w_body(w_ref, asrc_ref, adst_ref, o_ref):
    w3 = w_ref[...].reshape(EMB, HEADS, EMB)
    ws = jnp.sum(w3 * asrc_ref[...][None, :, :], axis=2)
    wd = jnp.sum(w3 * adst_ref[...][None, :, :], axis=2)
    o_ref[...] = jnp.concatenate([ws, wd], axis=1)


def _tc_prep_w(w, att_src, att_dst):
    return pl.pallas_call(
        _prep_w_body,
        out_shape=jax.ShapeDtypeStruct((EMB, 2 * HEADS), jnp.float32),
    )(w, att_src, att_dst)


def _prep_we_body(we_ref, ae_ref, o_ref):
    w3 = we_ref[...].reshape(EDGE_DIM, HEADS, EMB)
    o_ref[...] = jnp.sum(w3 * ae_ref[...][None, :, :], axis=2)


def _tc_prep_we(we, att_e):
    return pl.pallas_call(
        _prep_we_body,
        out_shape=jax.ShapeDtypeStruct((EDGE_DIM, HEADS), jnp.float32),
    )(we, att_e)


def _se_body(ea_ref, w1_ref, w2_ref, w3_ref, o1_ref, o2_ref, o3_ref,
             c1_ref, c2_ref, c3_ref):
    step = pl.program_id(0)
    ea = ea_ref[...]
    for w_ref, o_ref, c_ref in ((w1_ref, o1_ref, c1_ref),
                                (w2_ref, o2_ref, c2_ref),
                                (w3_ref, o3_ref, c3_ref)):
        v = jnp.dot(ea, w_ref[...], preferred_element_type=jnp.float32)
        o_ref[...] = v.T
        s = jnp.broadcast_to(jnp.sum(v, axis=0, keepdims=True), (8, HEADS))

        @pl.when(step == 0)
        def _():
            c_ref[...] = s

        @pl.when(step != 0)
        def _():
            c_ref[...] = c_ref[...] + s


def _tc_se(edge_attr, we1, we2, we3):
    eb = 128
    neb = E // eb
    return pl.pallas_call(
        _se_body,
        grid=(neb,),
        in_specs=[
            pl.BlockSpec((eb, EDGE_DIM), lambda i: (i, 0)),
            pl.BlockSpec((EDGE_DIM, HEADS), lambda i: (0, 0)),
            pl.BlockSpec((EDGE_DIM, HEADS), lambda i: (0, 0)),
            pl.BlockSpec((EDGE_DIM, HEADS), lambda i: (0, 0)),
        ],
        out_specs=[
            pl.BlockSpec((HEADS, eb), lambda i: (0, i)),
            pl.BlockSpec((HEADS, eb), lambda i: (0, i)),
            pl.BlockSpec((HEADS, eb), lambda i: (0, i)),
            pl.BlockSpec((8, HEADS), lambda i: (0, 0)),
            pl.BlockSpec((8, HEADS), lambda i: (0, 0)),
            pl.BlockSpec((8, HEADS), lambda i: (0, 0)),
        ],
        out_shape=[
            jax.ShapeDtypeStruct((HEADS, E), jnp.float32),
            jax.ShapeDtypeStruct((HEADS, E), jnp.float32),
            jax.ShapeDtypeStruct((HEADS, E), jnp.float32),
            jax.ShapeDtypeStruct((8, HEADS), jnp.float32),
            jax.ShapeDtypeStruct((8, HEADS), jnp.float32),
            jax.ShapeDtypeStruct((8, HEADS), jnp.float32),
        ],
    )(edge_attr, we1, we2, we3)


def _se_fill_body(c1_ref, c2_ref, c3_ref, o1_ref, o2_ref, o3_ref):
    col = lax.broadcasted_iota(jnp.int32, (HEADS, EP_A - E), 1)
    for c_ref, o_ref in ((c1_ref, o1_ref), (c2_ref, o2_ref), (c3_ref, o3_ref)):
        loopval = c_ref[...][0:1, 0:HEADS].T * (1.0 / E)
        o_ref[...] = jnp.where(col < (EP - E),
                               jnp.broadcast_to(loopval, (HEADS, EP_A - E)),
                               -1e30)


def _tc_se_fill(c1, c2, c3):
    sh = jax.ShapeDtypeStruct((HEADS, EP_A - E), jnp.float32)
    return pl.pallas_call(
        _se_fill_body,
        out_shape=[sh, sh, sh],
    )(c1, c2, c3)


def _bn_stats_body(m_ref, o_ref):
    step = pl.program_id(0)
    m = m_ref[...]
    s = jnp.sum(m, axis=0, keepdims=True)
    q = jnp.sum(m * m, axis=0, keepdims=True)
    blk = jnp.concatenate([s, q, jnp.zeros((6, EMB), jnp.float32)], axis=0)

    @pl.when(step == 0)
    def _():
        o_ref[...] = blk

    @pl.when(step != 0)
    def _():
        o_ref[...] = o_ref[...] + blk


def _tc_bn_stats(msg):
    return pl.pallas_call(
        _bn_stats_body,
        grid=(NRB,),
        in_specs=[pl.BlockSpec((RB, EMB), lambda i: (i, 0))],
        out_specs=pl.BlockSpec((8, EMB), lambda i: (0, 0)),
        out_shape=jax.ShapeDtypeStruct((8, EMB), jnp.float32),
    )(msg)


def _bn_apply_body(m_ref, hp_ref, st_ref, g_ref, bt_ref, o_ref):
    st = st_ref[...]
    mu = st[0:1, :] * (1.0 / N)
    var = st[1:2, :] * (1.0 / N) - mu * mu
    inv = lax.rsqrt(var + 1e-5)
    hn = (m_ref[...] - mu) * (inv * g_ref[...]) + bt_ref[...]
    o_ref[...] = hp_ref[...] + jnp.maximum(hn, 0.0)


def _tc_bn_apply(msg, h_prev, stats, gamma, beta):
    return pl.pallas_call(
        _bn_apply_body,
        grid=(NRB,),
        in_specs=[
            pl.BlockSpec((RB, EMB), lambda i: (i, 0)),
            pl.BlockSpec((RB, EMB), lambda i: (i, 0)),
            pl.BlockSpec((8, EMB), lambda i: (0, 0)),
            pl.BlockSpec((1, EMB), lambda i: (0, 0)),
            pl.BlockSpec((1, EMB), lambda i: (0, 0)),
        ],
        out_specs=pl.BlockSpec((RB, EMB), lambda i: (i, 0)),
        out_shape=jax.ShapeDtypeStruct((N, EMB), jnp.float32),
    )(msg, h_prev, stats, gamma.reshape(1, EMB), beta.reshape(1, EMB))


def _pool_body(h1_ref, h2_ref, h3_ref, b_ref, o1_ref, o2_ref, o3_ref):
    step = pl.program_id(0)
    bvec = b_ref[...][:, 0]
    oh = (lax.broadcasted_iota(jnp.int32, (B, RB), 0) == bvec[None, :])
    oh = oh.astype(jnp.float32)
    for h_ref, o_ref in ((h1_ref, o1_ref), (h2_ref, o2_ref), (h3_ref, o3_ref)):
        v = jnp.dot(oh, h_ref[...], preferred_element_type=jnp.float32)

        @pl.when(step == 0)
        def _():
            o_ref[...] = v

        @pl.when(step != 0)
        def _():
            o_ref[...] = o_ref[...] + v


def _tc_pool(h1, h2, h3, batch):
    sh = jax.ShapeDtypeStruct((B, EMB), jnp.float32)
    return pl.pallas_call(
        _pool_body,
        grid=(NRB,),
        in_specs=[
            pl.BlockSpec((RB, EMB), lambda i: (i, 0)),
            pl.BlockSpec((RB, EMB), lambda i: (i, 0)),
            pl.BlockSpec((RB, EMB), lambda i: (i, 0)),
            pl.BlockSpec((RB, 1), lambda i: (i, 0)),
        ],
        out_specs=[
            pl.BlockSpec((B, EMB), lambda i: (0, 0)),
            pl.BlockSpec((B, EMB), lambda i: (0, 0)),
            pl.BlockSpec((B, EMB), lambda i: (0, 0)),
        ],
        out_shape=[sh, sh, sh],
    )(h1, h2, h3, batch.reshape(N, 1))


def _final_body(p1_ref, p2_ref, p3_ref, wg_ref, bg_ref, o_ref):
    p1, p2, p3 = p1_ref[...], p2_ref[...], p3_ref[...]
    zs = jnp.concatenate([p1, p2, p3], axis=1)
    g = jnp.dot(zs, wg_ref[...], preferred_element_type=jnp.float32) + bg_ref[...]
    g = g - jnp.max(g, axis=1, keepdims=True)
    eg = jnp.exp(g)
    g = eg / jnp.sum(eg, axis=1, keepdims=True)
    o_ref[...] = p1 * g[:, 0:1] + p2 * g[:, 1:2] + p3 * g[:, 2:3]


def _tc_final(p1, p2, p3, wg, bg):
    return pl.pallas_call(
        _final_body,
        out_shape=jax.ShapeDtypeStruct((B, EMB), jnp.float32),
    )(p1, p2, p3, wg, bg.reshape(1, LAYERS))


def _sc_alpha_body(s_hbm, se_hbm, perm_hbm, srcs_hbm, dsts_hbm, starts_hbm,
                   alpha_hbm,
                   s_v, den_v, inv_v, perm_v, src_v, dst_v, se_v, al_v,
                   starts_v, sem):
    wid = lax.axis_index("s") * 2 + lax.axis_index("c")
    nb = wid * NPT
    pltpu.sync_copy(starts_hbm, starts_v)
    pltpu.sync_copy(s_hbm, s_v)
    wid16 = jnp.full((16,), wid, jnp.int32)
    estart = plsc.load_gather(starts_v, [wid16])[0]
    eend = plsc.load_gather(starts_v, [wid16 + 1])[0]
    iota = lax.iota(jnp.int32, 16)
    zero16 = jnp.zeros((16,), jnp.float32)

    def zero_den(i, _):
        den_v[pl.ds(i * 16, 16)] = zero16
        return 0

    lax.fori_loop(0, 16 * DEN // 16, zero_den, 0)

    def load_chunk(base):
        c1 = pltpu.async_copy(perm_hbm.at[pl.ds(base, CE_A)], perm_v, sem)
        c2 = pltpu.async_copy(srcs_hbm.at[pl.ds(base, CE_A)], src_v, sem)
        c3 = pltpu.async_copy(dsts_hbm.at[pl.ds(base, CE_A)], dst_v, sem)
        c1.wait()
        c2.wait()
        c3.wait()
        cs = [pltpu.async_copy(se_hbm.at[h].at[perm_v], se_v.at[h], sem)
              for h in range(HEADS)]
        for c in cs:
            c.wait()

    def groups():
        for g in range(CE_A // 16):
            src16 = src_v[pl.ds(g * 16, 16)]
            dst16 = dst_v[pl.ds(g * 16, 16)]
            dl16 = jnp.clip(dst16 - nb, 0, NPT - 1)
            sidx = src16 * 8
            didx = jnp.minimum(dst16, N - 1) * 8 + HEADS
            yield g, sidx, didx, dl16

    def ex_of(g, sidx, didx, h):
        ss = plsc.load_gather(s_v, [sidx + h])
        sd = plsc.load_gather(s_v, [didx + h])
        se16 = se_v[h, pl.ds(g * 16, 16)]
        a = ss + sd + se16
        a = jnp.where(a >= 0, a, a * 0.2)
        return jnp.exp(a)

    nchunk = (eend - estart) // CE_A

    def pass1(i, _):
        base = pl.multiple_of(estart + i * CE_A, CE_A)
        load_chunk(base)
        for g, sidx, didx, dl16 in groups():
            for h in range(HEADS):
                ex = ex_of(g, sidx, didx, h)
                plsc.addupdate_scatter(
                    den_v, [iota * DEN + dl16 * HEADS + h], ex)
        return 0

    lax.fori_loop(0, nchunk, pass1, 0)

    def red(j, _):
        acc = den_v[pl.ds(j * 16, 16)]
        for l in range(1, 16):
            acc = acc + den_v[pl.ds(l * DEN + j * 16, 16)]
        inv_v[pl.ds(j * 16, 16)] = 0.25 / (acc + 1e-16)
        return 0

    lax.fori_loop(0, DEN // 16, red, 0)

    def pass2(i, _):
        base = pl.multiple_of(estart + i * CE_A, CE_A)
        load_chunk(base)
        for g, sidx, didx, dl16 in groups():
            for h in range(HEADS):
                ex = ex_of(g, sidx, didx, h)
                inv16 = plsc.load_gather(inv_v, [dl16 * HEADS + h])
                al_v[h, pl.ds(g * 16, 16)] = ex * inv16
        for h in range(HEADS):
            pltpu.sync_copy(al_v.at[h], alpha_hbm.at[h].at[pl.ds(base, CE_A)])
        return 0

    lax.fori_loop(0, nchunk, pass2, 0)


def _sc_alpha(s_flat, se, perm, srcs, dsts, starts):
    f = functools.partial(
        pl.kernel,
        out_type=jax.ShapeDtypeStruct((HEADS, EPL), jnp.float32),
        mesh=_mesh(),
        compiler_params=pltpu.CompilerParams(
            needs_layout_passes=False, use_tc_tiling_on_sc=False),
        scratch_types=[
            pltpu.VMEM((N * 2 * HEADS,), jnp.float32),
            pltpu.VMEM((16 * DEN,), jnp.float32),
            pltpu.VMEM((DEN,), jnp.float32),
            pltpu.VMEM((CE_A,), jnp.int32),
            pltpu.VMEM((CE_A,), jnp.int32),
            pltpu.VMEM((CE_A,), jnp.int32),
            pltpu.VMEM((HEADS, CE_A), jnp.float32),
            pltpu.VMEM((HEADS, CE_A), jnp.float32),
            pltpu.VMEM((NTILE + 8,), jnp.int32),
            pltpu.SemaphoreType.DMA,
        ],
    )(_sc_alpha_body)
    return f(s_flat, se, perm, srcs, dsts, starts)


def _sc_msg_body(xh0_hbm, xh1_hbm, xh2_hbm, xh3_hbm, alpha_hbm,
                 srcs_hbm, dsts_hbm, starts_hbm,
                 msg_hbm,
                 acc_v, rows0_v, rows1_v, src0_v, src1_v, dst0_v, dst1_v,
                 al0_v, al1_v, starts_v, sem_idx, sem_rows):
    wid = lax.axis_index("s") * 2 + lax.axis_index("c")
    nb = wid * NPT
    pltpu.sync_copy(starts_hbm, starts_v)
    wid16 = jnp.full((16,), wid, jnp.int32)
    estart = plsc.load_gather(starts_v, [wid16])[0]
    eend = plsc.load_gather(starts_v, [wid16 + 1])[0]
    zero16 = jnp.zeros((16,), jnp.float32)

    def zero_acc(j, _):
        for k in range(EMB // 16):
            acc_v[j, pl.ds(k * 16, 16)] = zero16
        return 0

    lax.fori_loop(0, NPT, zero_acc, 0)

    nchunk = (eend - estart) // CE_B
    xhs = (xh0_hbm, xh1_hbm, xh2_hbm, xh3_hbm)
    srcs = (src0_v, src1_v)
    dsts = (dst0_v, dst1_v)
    als = (al0_v, al1_v)
    rows = (rows0_v, rows1_v)

    def base_of(i):
        return pl.multiple_of(estart + i * CE_B, CE_B)

    def idx_copies(i, s):
        base = base_of(i)
        cps = [pltpu.make_async_copy(srcs_hbm.at[pl.ds(base, CE_B)],
                                     srcs[s], sem_idx),
               pltpu.make_async_copy(dsts_hbm.at[pl.ds(base, CE_B)],
                                     dsts[s].at[pl.ds(0, CE_B)], sem_idx)]
        for h in range(HEADS):
            cps.append(pltpu.make_async_copy(
                alpha_hbm.at[h].at[pl.ds(base, CE_B)],
                als[s].at[pl.ds(h * CE_B, CE_B)], sem_idx))
        return cps

    def start_idx(i, s):
        for c in idx_copies(i, s):
            c.start()

    def drain_idx(s):
        for c in idx_copies(0, s):
            c.wait()

    def rows_copy(h, src_slot, r):
        return pltpu.make_async_copy(xhs[h].at[srcs[src_slot]], rows[r],
                                     sem_rows)

    def compute(h, p, r):
        def grp(g, _):
            off = pl.multiple_of(g * 8, 8)
            dl16 = dsts[p][pl.ds(off, 16)] - nb
            a16 = als[p][pl.ds(h * CE_B + off, 16)]
            for j in range(8):
                dl = dl16[j]
                a = a16[j]
                e = off + j
                for k in range(EMB // 16):
                    sl = pl.ds(k * 16, 16)
                    plsc.addupdate(acc_v.at[dl, sl], a * rows[r][e, sl])
            return 0

        lax.fori_loop(0, CE_B // 8, grp, 0)

    start_idx(0, 0)
    drain_idx(0)
    rows_copy(0, 0, 0).start()
    start_idx(1, 1)

    def pair(jj, _):
        for par in range(2):
            i = 2 * jj + par
            rows_copy(0, par, 0).wait()
            rows_copy(1, par, 1).start()
            compute(0, par, 0)
            rows_copy(1, par, 1).wait()
            rows_copy(2, par, 0).start()
            compute(1, par, 1)
            rows_copy(2, par, 0).wait()
            rows_copy(3, par, 1).start()
            compute(2, par, 0)
            drain_idx(1 - par)
            rows_copy(0, 1 - par, 0).start()
            rows_copy(3, par, 1).wait()
            compute(3, par, 1)
            start_idx(jnp.minimum(i + 2, nchunk - 1), par)
        return 0

    lax.fori_loop(0, nchunk // 2, pair, 0)
    drain_idx(0)
    rows_copy(0, 0, 0).wait()
    pltpu.sync_copy(acc_v, msg_hbm.at[pl.ds(pl.multiple_of(nb, NPT), NPT)])


def _sc_msg(xh0, xh1, xh2, xh3, alpha, srcs, dsts, starts):
    f = functools.partial(
        pl.kernel,
        out_type=jax.ShapeDtypeStruct((NPAD, EMB), jnp.float32),
        mesh=_mesh(),
        compiler_params=pltpu.CompilerParams(
            needs_layout_passes=False, use_tc_tiling_on_sc=False),
        scratch_types=[
            pltpu.VMEM((NPT, EMB), jnp.float32),
            pltpu.VMEM((CE_B, EMB), jnp.float32),
            pltpu.VMEM((CE_B, EMB), jnp.float32),
            pltpu.VMEM((CE_B,), jnp.int32),
            pltpu.VMEM((CE_B,), jnp.int32),
            pltpu.VMEM((CE_B + 16,), jnp.int32),
            pltpu.VMEM((CE_B + 16,), jnp.int32),
            pltpu.VMEM((HEADS * CE_B + 16,), jnp.float32),
            pltpu.VMEM((HEADS * CE_B + 16,), jnp.float32),
            pltpu.VMEM((NTILE + 8,), jnp.int32),
            pltpu.SemaphoreType.DMA,
            pltpu.SemaphoreType.DMA,
        ],
    )(_sc_msg_body)
    return f(xh0, xh1, xh2, xh3, alpha, srcs, dsts, starts)


def kernel(x, edge_index, edge_attr, batch, params):
    p = params

    loops = jnp.arange(N, dtype=jnp.int32)
    src_all = jnp.concatenate([edge_index[0], loops])
    dst_all = jnp.concatenate([edge_index[1], loops])
    npad_e = EP - (E + N)
    src_p = jnp.concatenate([src_all, jnp.zeros((npad_e,), jnp.int32)])
    dst_p = jnp.concatenate([dst_all, jnp.full((npad_e,), NPAD - 1, jnp.int32)])
    tid = dst_p // NPT
    onehot = (tid[:, None] ==
              jnp.arange(NTILE, dtype=jnp.int32)[None, :]).astype(jnp.int32)
    csum = jnp.cumsum(onehot, axis=0)
    cnt = csum[-1]
    rank = jnp.take_along_axis(csum, tid[:, None], axis=1)[:, 0] - 1
    cap = ((cnt + CE_A - 1) // CE_A) * CE_A
    astart = jnp.concatenate(
        [jnp.zeros((1,), jnp.int32), jnp.cumsum(cap).astype(jnp.int32)])
    pos = astart[tid] + rank
    perm_f = jnp.full((EPL,), EP, jnp.int32).at[pos].set(
        jnp.arange(EP, dtype=jnp.int32))
    srcs_f = jnp.zeros((EPL,), jnp.int32).at[pos].set(src_p)
    slot_tid = jnp.searchsorted(
        astart[1:], jnp.arange(EPL, dtype=jnp.int32), side='right')
    dinit = jnp.clip((slot_tid + 1) * NPT - 1, 0, NPAD - 1).astype(jnp.int32)
    dsts_f = dinit.at[pos].set(dst_p)
    starts = jnp.concatenate(
        [astart, jnp.full((NTILE + 8 - (NTILE + 1),), astart[-1], jnp.int32)])

    ws_l = [_tc_prep_w(lp['W'], lp['att_src'], lp['att_dst'])
            for lp in p['layers']]
    we_l = [_tc_prep_we(lp['We'], lp['att_e']) for lp in p['layers']]
    se1, se2, se3, c1, c2, c3 = _tc_se(edge_attr, *we_l)
    f1, f2, f3 = _tc_se_fill(c1, c2, c3)
    se_full = [jnp.concatenate([t, f], axis=1)
               for t, f in ((se1, f1), (se2, f2), (se3, f3))]

    h = _tc_h0(x, p['W0'], p['b0'])

    outs = []
    for l, lp in enumerate(p['layers']):
        xh4 = _tc_xh(h, lp['W'])
        s = _tc_scores(h, ws_l[l])
        alpha = _sc_alpha(s.reshape(N * 2 * HEADS), se_full[l],
                          perm_f, srcs_f, dsts_f, starts)
        msg = _sc_msg(xh4[0], xh4[1], xh4[2], xh4[3], alpha,
                      srcs_f, dsts_f, starts)
        stats = _tc_bn_stats(msg)
        h = _tc_bn_apply(msg, h, stats, lp['gamma'], lp['beta'])
        outs.append(h)

    p1, p2, p3 = _tc_pool(outs[0], outs[1], outs[2], batch)
    z = _tc_final(p1, p2, p3, p['Wg'], p['bg'])
    return (z, outs[-1])


# speedup vs baseline: 7.8343x; 1.1416x over previous
import functools

import jax
import jax.numpy as jnp
from jax import lax
from jax.experimental import pallas as pl
from jax.experimental.pallas import tpu as pltpu
from jax.experimental.pallas import tpu_sc as plsc

N = 10000
E = 160000
IN_DIM = 128
EMB = 256
HEADS = 4
LAYERS = 3
EDGE_DIM = 16
B = 64

NTILE = 32
NPT = 320
NPAD = NTILE * NPT
EP = 170240
EP_A = EP + 128
EPL = EP + 8192
RB = 80
NRB = N // RB
CE_A = 128
CE_B = 64
DEN = NPT * HEADS


def _mesh():
    return plsc.VectorSubcoreMesh(
        core_axis_name="c", subcore_axis_name="s", num_cores=2, num_subcores=16)


def _h0_body(x_ref, w_ref, b_ref, o_ref):
    v = jnp.dot(x_ref[...], w_ref[...], preferred_element_type=jnp.float32)
    o_ref[...] = jnp.maximum(v + b_ref[...], 0.0)


def _tc_h0(x, w0, b0):
    return pl.pallas_call(
        _h0_body,
        grid=(NRB,),
        in_specs=[
            pl.BlockSpec((RB, IN_DIM), lambda i: (i, 0)),
            pl.BlockSpec((IN_DIM, EMB), lambda i: (0, 0)),
            pl.BlockSpec((1, EMB), lambda i: (0, 0)),
        ],
        out_specs=pl.BlockSpec((RB, EMB), lambda i: (i, 0)),
        out_shape=jax.ShapeDtypeStruct((N, EMB), jnp.float32),
    )(x, w0, b0.reshape(1, EMB))


def _xh_body(h_ref, w_ref, o_ref):
    o_ref[0] = jnp.dot(h_ref[...], w_ref[...], preferred_element_type=jnp.float32)


def _tc_xh(h, w):
    return pl.pallas_call(
        _xh_body,
        grid=(HEADS, NRB),
        in_specs=[
            pl.BlockSpec((RB, EMB), lambda hd, rb: (rb, 0)),
            pl.BlockSpec((EMB, EMB), lambda hd, rb: (0, hd)),
        ],
        out_specs=pl.BlockSpec((1, RB, EMB), lambda hd, rb: (hd, rb, 0)),
        out_shape=jax.ShapeDtypeStruct((HEADS, N, EMB), jnp.float32),
    )(h, w)


def _s_body(h_ref, ws_ref, o_ref):
    o_ref[...] = jnp.dot(h_ref[...], ws_ref[...], preferred_element_type=jnp.float32)


def _tc_scores(h, ws):
    return pl.pallas_call(
        _s_body,
        grid=(NRB,),
        in_specs=[
            pl.BlockSpec((RB, EMB), lambda i: (i, 0)),
            pl.BlockSpec((EMB, 2 * HEADS), lambda i: (0, 0)),
        ],
        out_specs=pl.BlockSpec((RB, 2 * HEADS), lambda i: (i, 0)),
        out_shape=jax.ShapeDtypeStruct((N, 2 * HEADS), jnp.float32),
    )(h, ws)


def _prep_w_body(w_ref, asrc_ref, adst_ref, o_ref):
    w3 = w_ref[...].reshape(EMB, HEADS, EMB)
    ws = jnp.sum(w3 * asrc_ref[...][None, :, :], axis=2)
    wd = jnp.sum(w3 * adst_ref[...][None, :, :], axis=2)
    o_ref[...] = jnp.concatenate([ws, wd], axis=1)


def _tc_prep_w(w, att_src, att_dst):
    return pl.pallas_call(
        _prep_w_body,
        out_shape=jax.ShapeDtypeStruct((EMB, 2 * HEADS), jnp.float32),
    )(w, att_src, att_dst)


def _prep_we_body(we_ref, ae_ref, o_ref):
    w3 = we_ref[...].reshape(EDGE_DIM, HEADS, EMB)
    o_ref[...] = jnp.sum(w3 * ae_ref[...][None, :, :], axis=2)


def _tc_prep_we(we, att_e):
    return pl.pallas_call(
        _prep_we_body,
        out_shape=jax.ShapeDtypeStruct((EDGE_DIM, HEADS), jnp.float32),
    )(we, att_e)


def _se_body(ea_ref, w1_ref, w2_ref, w3_ref, o1_ref, o2_ref, o3_ref,
             c1_ref, c2_ref, c3_ref):
    step = pl.program_id(0)
    ea = ea_ref[...]
    for w_ref, o_ref, c_ref in ((w1_ref, o1_ref, c1_ref),
                                (w2_ref, o2_ref, c2_ref),
                                (w3_ref, o3_ref, c3_ref)):
        v = jnp.dot(ea, w_ref[...], preferred_element_type=jnp.float32)
        o_ref[...] = v.T
        s = jnp.broadcast_to(jnp.sum(v, axis=0, keepdims=True), (8, HEADS))

        @pl.when(step == 0)
        def _():
            c_ref[...] = s

        @pl.when(step != 0)
        def _():
            c_ref[...] = c_ref[...] + s


def _tc_se(edge_attr, we1, we2, we3):
    eb = 128
    neb = E // eb
    return pl.pallas_call(
        _se_body,
        grid=(neb,),
        in_specs=[
            pl.BlockSpec((eb, EDGE_DIM), lambda i: (i, 0)),
            pl.BlockSpec((EDGE_DIM, HEADS), lambda i: (0, 0)),
            pl.BlockSpec((EDGE_DIM, HEADS), lambda i: (0, 0)),
            pl.BlockSpec((EDGE_DIM, HEADS), lambda i: (0, 0)),
        ],
        out_specs=[
            pl.BlockSpec((HEADS, eb), lambda i: (0, i)),
            pl.BlockSpec((HEADS, eb), lambda i: (0, i)),
            pl.BlockSpec((HEADS, eb), lambda i: (0, i)),
            pl.BlockSpec((8, HEADS), lambda i: (0, 0)),
            pl.BlockSpec((8, HEADS), lambda i: (0, 0)),
            pl.BlockSpec((8, HEADS), lambda i: (0, 0)),
        ],
        out_shape=[
            jax.ShapeDtypeStruct((HEADS, E), jnp.float32),
            jax.ShapeDtypeStruct((HEADS, E), jnp.float32),
            jax.ShapeDtypeStruct((HEADS, E), jnp.float32),
            jax.ShapeDtypeStruct((8, HEADS), jnp.float32),
            jax.ShapeDtypeStruct((8, HEADS), jnp.float32),
            jax.ShapeDtypeStruct((8, HEADS), jnp.float32),
        ],
    )(edge_attr, we1, we2, we3)


def _se_fill_body(c1_ref, c2_ref, c3_ref, o1_ref, o2_ref, o3_ref):
    col = lax.broadcasted_iota(jnp.int32, (HEADS, EP_A - E), 1)
    for c_ref, o_ref in ((c1_ref, o1_ref), (c2_ref, o2_ref), (c3_ref, o3_ref)):
        loopval = c_ref[...][0:1, 0:HEADS].T * (1.0 / E)
        o_ref[...] = jnp.where(col < (EP - E),
                               jnp.broadcast_to(loopval, (HEADS, EP_A - E)),
                               -1e30)


def _tc_se_fill(c1, c2, c3):
    sh = jax.ShapeDtypeStruct((HEADS, EP_A - E), jnp.float32)
    return pl.pallas_call(
        _se_fill_body,
        out_shape=[sh, sh, sh],
    )(c1, c2, c3)


def _bn_stats_body(m_ref, o_ref):
    step = pl.program_id(0)
    m = m_ref[...]
    s = jnp.sum(m, axis=0, keepdims=True)
    q = jnp.sum(m * m, axis=0, keepdims=True)
    blk = jnp.concatenate([s, q, jnp.zeros((6, EMB), jnp.float32)], axis=0)

    @pl.when(step == 0)
    def _():
        o_ref[...] = blk

    @pl.when(step != 0)
    def _():
        o_ref[...] = o_ref[...] + blk


def _tc_bn_stats(msg):
    return pl.pallas_call(
        _bn_stats_body,
        grid=(NRB,),
        in_specs=[pl.BlockSpec((RB, EMB), lambda i: (i, 0))],
        out_specs=pl.BlockSpec((8, EMB), lambda i: (0, 0)),
        out_shape=jax.ShapeDtypeStruct((8, EMB), jnp.float32),
    )(msg)


def _bn_apply_body(m_ref, hp_ref, st_ref, g_ref, bt_ref, o_ref):
    st = st_ref[...]
    mu = st[0:1, :] * (1.0 / N)
    var = st[1:2, :] * (1.0 / N) - mu * mu
    inv = lax.rsqrt(var + 1e-5)
    hn = (m_ref[...] - mu) * (inv * g_ref[...]) + bt_ref[...]
    o_ref[...] = hp_ref[...] + jnp.maximum(hn, 0.0)


def _tc_bn_apply(msg, h_prev, stats, gamma, beta):
    return pl.pallas_call(
        _bn_apply_body,
        grid=(NRB,),
        in_specs=[
            pl.BlockSpec((RB, EMB), lambda i: (i, 0)),
            pl.BlockSpec((RB, EMB), lambda i: (i, 0)),
            pl.BlockSpec((8, EMB), lambda i: (0, 0)),
            pl.BlockSpec((1, EMB), lambda i: (0, 0)),
            pl.BlockSpec((1, EMB), lambda i: (0, 0)),
        ],
        out_specs=pl.BlockSpec((RB, EMB), lambda i: (i, 0)),
        out_shape=jax.ShapeDtypeStruct((N, EMB), jnp.float32),
    )(msg, h_prev, stats, gamma.reshape(1, EMB), beta.reshape(1, EMB))


def _pool_body(h1_ref, h2_ref, h3_ref, b_ref, o1_ref, o2_ref, o3_ref):
    step = pl.program_id(0)
    bvec = b_ref[...][:, 0]
    oh = (lax.broadcasted_iota(jnp.int32, (B, RB), 0) == bvec[None, :])
    oh = oh.astype(jnp.float32)
    for h_ref, o_ref in ((h1_ref, o1_ref), (h2_ref, o2_ref), (h3_ref, o3_ref)):
        v = jnp.dot(oh, h_ref[...], preferred_element_type=jnp.float32)

        @pl.when(step == 0)
        def _():
            o_ref[...] = v

        @pl.when(step != 0)
        def _():
            o_ref[...] = o_ref[...] + v


def _tc_pool(h1, h2, h3, batch):
    sh = jax.ShapeDtypeStruct((B, EMB), jnp.float32)
    return pl.pallas_call(
        _pool_body,
        grid=(NRB,),
        in_specs=[
            pl.BlockSpec((RB, EMB), lambda i: (i, 0)),
            pl.BlockSpec((RB, EMB), lambda i: (i, 0)),
            pl.BlockSpec((RB, EMB), lambda i: (i, 0)),
            pl.BlockSpec((RB, 1), lambda i: (i, 0)),
        ],
        out_specs=[
            pl.BlockSpec((B, EMB), lambda i: (0, 0)),
            pl.BlockSpec((B, EMB), lambda i: (0, 0)),
            pl.BlockSpec((B, EMB), lambda i: (0, 0)),
        ],
        out_shape=[sh, sh, sh],
    )(h1, h2, h3, batch.reshape(N, 1))


def _final_body(p1_ref, p2_ref, p3_ref, wg_ref, bg_ref, o_ref):
    p1, p2, p3 = p1_ref[...], p2_ref[...], p3_ref[...]
    zs = jnp.concatenate([p1, p2, p3], axis=1)
    g = jnp.dot(zs, wg_ref[...], preferred_element_type=jnp.float32) + bg_ref[...]
    g = g - jnp.max(g, axis=1, keepdims=True)
    eg = jnp.exp(g)
    g = eg / jnp.sum(eg, axis=1, keepdims=True)
    o_ref[...] = p1 * g[:, 0:1] + p2 * g[:, 1:2] + p3 * g[:, 2:3]


def _tc_final(p1, p2, p3, wg, bg):
    return pl.pallas_call(
        _final_body,
        out_shape=jax.ShapeDtypeStruct((B, EMB), jnp.float32),
    )(p1, p2, p3, wg, bg.reshape(1, LAYERS))


def _sc_alpha_body(s_hbm, se_hbm, perm_hbm, srcs_hbm, dsts_hbm, starts_hbm,
                   alpha_hbm,
                   s_v, den_v, inv_v, perm_v, src_v, dst_v, se_v, al_v,
                   starts_v, sem):
    wid = lax.axis_index("s") * 2 + lax.axis_index("c")
    nb = wid * NPT
    pltpu.sync_copy(starts_hbm, starts_v)
    pltpu.sync_copy(s_hbm, s_v)
    wid16 = jnp.full((16,), wid, jnp.int32)
    estart = plsc.load_gather(starts_v, [wid16])[0]
    eend = plsc.load_gather(starts_v, [wid16 + 1])[0]
    iota = lax.iota(jnp.int32, 16)
    zero16 = jnp.zeros((16,), jnp.float32)

    def zero_den(i, _):
        den_v[pl.ds(i * 16, 16)] = zero16
        return 0

    lax.fori_loop(0, 16 * DEN // 16, zero_den, 0)

    def load_chunk(base):
        pltpu.async_copy(perm_hbm.at[pl.ds(base, CE_A)], perm_v, sem).wait()
        cs = [pltpu.async_copy(srcs_hbm.at[perm_v], src_v, sem),
              pltpu.async_copy(dsts_hbm.at[perm_v], dst_v, sem)]
        cs += [pltpu.async_copy(se_hbm.at[h].at[perm_v], se_v.at[h], sem)
               for h in range(HEADS)]
        for c in cs:
            c.wait()

    def groups():
        for g in range(CE_A // 16):
            src16 = src_v[pl.ds(g * 16, 16)]
            dst16 = dst_v[pl.ds(g * 16, 16)]
            dl16 = jnp.clip(dst16 - nb, 0, NPT - 1)
            sidx = src16 * 8
            didx = jnp.minimum(dst16, N - 1) * 8 + HEADS
            yield g, sidx, didx, dl16

    def ex_of(g, sidx, didx, h):
        ss = plsc.load_gather(s_v, [sidx + h])
        sd = plsc.load_gather(s_v, [didx + h])
        se16 = se_v[h, pl.ds(g * 16, 16)]
        a = ss + sd + se16
        a = jnp.where(a >= 0, a, a * 0.2)
        return jnp.exp(a)

    nchunk = (eend - estart) // CE_A

    def pass1(i, _):
        base = pl.multiple_of(estart + i * CE_A, CE_A)
        load_chunk(base)
        for g, sidx, didx, dl16 in groups():
            for h in range(HEADS):
                ex = ex_of(g, sidx, didx, h)
                plsc.addupdate_scatter(
                    den_v, [iota * DEN + dl16 * HEADS + h], ex)
        return 0

    lax.fori_loop(0, nchunk, pass1, 0)

    def red(j, _):
        acc = den_v[pl.ds(j * 16, 16)]
        for l in range(1, 16):
            acc = acc + den_v[pl.ds(l * DEN + j * 16, 16)]
        inv_v[pl.ds(j * 16, 16)] = 0.25 / (acc + 1e-16)
        return 0

    lax.fori_loop(0, DEN // 16, red, 0)

    def pass2(i, _):
        base = pl.multiple_of(estart + i * CE_A, CE_A)
        load_chunk(base)
        for g, sidx, didx, dl16 in groups():
            for h in range(HEADS):
                ex = ex_of(g, sidx, didx, h)
                inv16 = plsc.load_gather(inv_v, [dl16 * HEADS + h])
                al_v[h, pl.ds(g * 16, 16)] = ex * inv16
        for h in range(HEADS):
            pltpu.sync_copy(al_v.at[h], alpha_hbm.at[h].at[pl.ds(base, CE_A)])
        return 0

    lax.fori_loop(0, nchunk, pass2, 0)


def _sc_alpha(s_flat, se, perm, srcs, dsts, starts):
    f = functools.partial(
        pl.kernel,
        out_type=jax.ShapeDtypeStruct((HEADS, EPL), jnp.float32),
        mesh=_mesh(),
        compiler_params=pltpu.CompilerParams(
            needs_layout_passes=False, use_tc_tiling_on_sc=False),
        scratch_types=[
            pltpu.VMEM((N * 2 * HEADS,), jnp.float32),
            pltpu.VMEM((16 * DEN,), jnp.float32),
            pltpu.VMEM((DEN,), jnp.float32),
            pltpu.VMEM((CE_A,), jnp.int32),
            pltpu.VMEM((CE_A,), jnp.int32),
            pltpu.VMEM((CE_A,), jnp.int32),
            pltpu.VMEM((HEADS, CE_A), jnp.float32),
            pltpu.VMEM((HEADS, CE_A), jnp.float32),
            pltpu.VMEM((NTILE + 8,), jnp.int32),
            pltpu.SemaphoreType.DMA,
        ],
    )(_sc_alpha_body)
    return f(s_flat, se, perm, srcs, dsts, starts)


def _sc_msg_body(xh0_hbm, xh1_hbm, xh2_hbm, xh3_hbm, alpha_hbm,
                 perm_hbm, srcs_hbm, dsts_hbm, starts_hbm,
                 msg_hbm,
                 acc_v, rows0_v, rows1_v, perm0_v, perm1_v,
                 src0_v, src1_v, dst0_v, dst1_v,
                 al0_v, al1_v, starts_v, sem_perm, sem_idx, sem_rows):
    wid = lax.axis_index("s") * 2 + lax.axis_index("c")
    nb = wid * NPT
    pltpu.sync_copy(starts_hbm, starts_v)
    wid16 = jnp.full((16,), wid, jnp.int32)
    estart = plsc.load_gather(starts_v, [wid16])[0]
    eend = plsc.load_gather(starts_v, [wid16 + 1])[0]
    zero16 = jnp.zeros((16,), jnp.float32)

    def zero_acc(j, _):
        for k in range(EMB // 16):
            acc_v[j, pl.ds(k * 16, 16)] = zero16
        return 0

    lax.fori_loop(0, NPT, zero_acc, 0)

    nchunk = (eend - estart) // CE_B
    xhs = (xh0_hbm, xh1_hbm, xh2_hbm, xh3_hbm)
    perms = (perm0_v, perm1_v)
    srcs = (src0_v, src1_v)
    dsts = (dst0_v, dst1_v)
    als = (al0_v, al1_v)
    rows = (rows0_v, rows1_v)

    def base_of(i):
        return pl.multiple_of(estart + i * CE_B, CE_B)

    def perm_copy(i, s):
        return pltpu.make_async_copy(perm_hbm.at[pl.ds(base_of(i), CE_B)],
                                     perms[s], sem_perm)

    def idx_copies(i, s):
        base = base_of(i)
        cps = [pltpu.make_async_copy(srcs_hbm.at[perms[s]],
                                     srcs[s], sem_idx),
               pltpu.make_async_copy(dsts_hbm.at[perms[s]],
                                     dsts[s].at[pl.ds(0, CE_B)], sem_idx)]
        for h in range(HEADS):
            cps.append(pltpu.make_async_copy(
                alpha_hbm.at[h].at[pl.ds(base, CE_B)],
                als[s].at[pl.ds(h * CE_B, CE_B)], sem_idx))
        return cps

    def start_idx(i, s):
        for c in idx_copies(i, s):
            c.start()

    def drain_idx(s):
        for c in idx_copies(0, s):
            c.wait()

    def rows_copy(h, src_slot, r):
        return pltpu.make_async_copy(xhs[h].at[srcs[src_slot]], rows[r],
                                     sem_rows)

    def compute(h, p, r):
        def grp(g, _):
            off = pl.multiple_of(g * 8, 8)
            dl16 = jnp.clip(dsts[p][pl.ds(off, 16)] - nb, 0, NPT - 1)
            a16 = als[p][pl.ds(h * CE_B + off, 16)]
            for j in range(8):
                dl = dl16[j]
                a = a16[j]
                e = off + j
                for k in range(EMB // 16):
                    sl = pl.ds(k * 16, 16)
                    plsc.addupdate(acc_v.at[dl, sl], a * rows[r][e, sl])
            return 0

        lax.fori_loop(0, CE_B // 8, grp, 0)

    perm_copy(0, 0).start()
    perm_copy(0, 0).wait()
    start_idx(0, 0)
    perm_copy(1, 1).start()
    drain_idx(0)
    rows_copy(0, 0, 0).start()

    def pair(jj, _):
        for par in range(2):
            i = 2 * jj + par
            rows_copy(0, par, 0).wait()
            rows_copy(1, par, 1).start()
            compute(0, par, 0)
            rows_copy(1, par, 1).wait()
            rows_copy(2, par, 0).start()
            compute(1, par, 1)
            rows_copy(2, par, 0).wait()
            rows_copy(3, par, 1).start()
            compute(2, par, 0)
            perm_copy(0, 1 - par).wait()
            start_idx(jnp.minimum(i + 1, nchunk - 1), 1 - par)
            drain_idx(1 - par)
            rows_copy(0, 1 - par, 0).start()
            rows_copy(3, par, 1).wait()
            compute(3, par, 1)
            perm_copy(jnp.minimum(i + 2, nchunk - 1), par).start()
        return 0

    lax.fori_loop(0, nchunk // 2, pair, 0)
    perm_copy(0, 0).wait()
    rows_copy(0, 0, 0).wait()
    pltpu.sync_copy(acc_v, msg_hbm.at[pl.ds(pl.multiple_of(nb, NPT), NPT)])


def _sc_msg(xh0, xh1, xh2, xh3, alpha, perm, srcs, dsts, starts):
    f = functools.partial(
        pl.kernel,
        out_type=jax.ShapeDtypeStruct((NPAD, EMB), jnp.float32),
        mesh=_mesh(),
        compiler_params=pltpu.CompilerParams(
            needs_layout_passes=False, use_tc_tiling_on_sc=False),
        scratch_types=[
            pltpu.VMEM((NPT, EMB), jnp.float32),
            pltpu.VMEM((CE_B, EMB), jnp.float32),
            pltpu.VMEM((CE_B, EMB), jnp.float32),
            pltpu.VMEM((CE_B,), jnp.int32),
            pltpu.VMEM((CE_B,), jnp.int32),
            pltpu.VMEM((CE_B,), jnp.int32),
            pltpu.VMEM((CE_B,), jnp.int32),
            pltpu.VMEM((CE_B + 16,), jnp.int32),
            pltpu.VMEM((CE_B + 16,), jnp.int32),
            pltpu.VMEM((HEADS * CE_B + 16,), jnp.float32),
            pltpu.VMEM((HEADS * CE_B + 16,), jnp.float32),
            pltpu.VMEM((NTILE + 8,), jnp.int32),
            pltpu.SemaphoreType.DMA,
            pltpu.SemaphoreType.DMA,
            pltpu.SemaphoreType.DMA,
        ],
    )(_sc_msg_body)
    return f(xh0, xh1, xh2, xh3, alpha, perm, srcs, dsts, starts)


def kernel(x, edge_index, edge_attr, batch, params):
    p = params

    loops = jnp.arange(N, dtype=jnp.int32)
    src_all = jnp.concatenate([edge_index[0], loops])
    dst_all = jnp.concatenate([edge_index[1], loops])
    npad_e = EP - (E + N)
    src_p = jnp.concatenate([src_all, jnp.zeros((npad_e,), jnp.int32)])
    dst_p = jnp.concatenate([dst_all, jnp.full((npad_e,), NPAD - 1, jnp.int32)])
    tid = dst_p // NPT
    onehot = (tid[:, None] ==
              jnp.arange(NTILE, dtype=jnp.int32)[None, :]).astype(jnp.int32)
    csum = jnp.cumsum(onehot, axis=0)
    cnt = csum[-1]
    rank = jnp.sum(onehot * (csum - 1), axis=1)
    cap = ((cnt + CE_A - 1) // CE_A) * CE_A
    astart = jnp.concatenate(
        [jnp.zeros((1,), jnp.int32), jnp.cumsum(cap).astype(jnp.int32)])
    pos = astart[tid] + rank
    perm_f = jnp.full((EPL,), EP, jnp.int32).at[pos].set(
        jnp.arange(EP, dtype=jnp.int32))
    src_p2 = jnp.concatenate([src_p, jnp.zeros((8,), jnp.int32)])
    dst_p2 = jnp.concatenate([dst_p, jnp.full((8,), NPAD - 1, jnp.int32)])
    starts = jnp.concatenate(
        [astart, jnp.full((NTILE + 8 - (NTILE + 1),), astart[-1], jnp.int32)])

    ws_l = [_tc_prep_w(lp['W'], lp['att_src'], lp['att_dst'])
            for lp in p['layers']]
    we_l = [_tc_prep_we(lp['We'], lp['att_e']) for lp in p['layers']]
    se1, se2, se3, c1, c2, c3 = _tc_se(edge_attr, *we_l)
    f1, f2, f3 = _tc_se_fill(c1, c2, c3)
    se_full = [jnp.concatenate([t, f], axis=1)
               for t, f in ((se1, f1), (se2, f2), (se3, f3))]

    h = _tc_h0(x, p['W0'], p['b0'])

    outs = []
    for l, lp in enumerate(p['layers']):
        xh4 = _tc_xh(h, lp['W'])
        s = _tc_scores(h, ws_l[l])
        alpha = _sc_alpha(s.reshape(N * 2 * HEADS), se_full[l],
                          perm_f, src_p2, dst_p2, starts)
        msg = _sc_msg(xh4[0], xh4[1], xh4[2], xh4[3], alpha,
                      perm_f, src_p2, dst_p2, starts)
        stats = _tc_bn_stats(msg)
        h = _tc_bn_apply(msg, h, stats, lp['gamma'], lp['beta'])
        outs.append(h)

    p1, p2, p3 = _tc_pool(outs[0], outs[1], outs[2], batch)
    z = _tc_final(p1, p2, p3, p['Wg'], p['bg'])
    return (z, outs[-1])
